# baseline (speedup 1.0000x reference)
import functools

import jax
import jax.numpy as jnp
from jax import lax
from jax.experimental import pallas as pl
from jax.experimental.pallas import tpu as pltpu
from jax.experimental.pallas import tpu_sc as plsc

N = 10000
NP = 10240
E = 320000
NC = 2
NS = 16
NW = NC * NS
CH = 128
RW = E // CH
NB = 4
MAXC = RW // NW + 1
NG = 20
RPT = NP // NS

_SC_PARAMS = pltpu.CompilerParams(use_tc_tiling_on_sc=False,
                                  needs_layout_passes=False)


def _worker_span(wid):
    lo = wid * RW // NW
    hi = (wid + 1) * RW // NW
    return lo, hi - lo


def _rsqrt16(d):
    i = plsc.bitcast(d, jnp.int32)
    y = plsc.bitcast(0x5F3759DF - lax.shift_right_logical(i, 1), jnp.float32)
    for _ in range(3):
        y = y * (1.5 - 0.5 * d * y * y)
    return y


def _edge_pipeline(sidx, didx, rows, gsem, ssem, g_sh, acc_sh, m):
    for b in range(NB):
        pltpu.async_copy(g_sh.at[sidx.at[b]], rows[b], gsem[b])

    def group(jj, carry):
        for b in range(NB):
            j = jj * NB + b

            @pl.when(j < m)
            def _():
                pltpu.make_async_copy(g_sh.at[sidx.at[j]], rows[b],
                                      gsem[b]).wait()
                pltpu.async_copy(rows[b], acc_sh.at[didx.at[j]], ssem[b],
                                 add=True)

        for b in range(NB):
            j = jj * NB + b

            @pl.when(j < m)
            def _():
                pltpu.make_async_copy(rows[b], acc_sh.at[didx.at[j]],
                                      ssem[b]).wait()

            @pl.when(j + NB < m)
            def _():
                pltpu.async_copy(g_sh.at[sidx.at[j + NB]], rows[b],
                                 gsem[b])
        return carry

    lax.fori_loop(0, NG, group, 0)


def _sc_degree():
    mesh = plsc.VectorSubcoreMesh(core_axis_name="c", subcore_axis_name="s")

    @functools.partial(
        pl.kernel,
        mesh=mesh,
        out_type=jax.ShapeDtypeStruct((NC, NP, 16), jnp.float32),
        compiler_params=_SC_PARAMS,
        scratch_types=[
            pltpu.VMEM((MAXC, CH), jnp.int32),
            pltpu.VMEM((CH, 16), jnp.float32),
            [pltpu.SemaphoreType.DMA for _ in range(NB)],
            pltpu.VMEM_SHARED((NP, 16), jnp.float32),
        ],
    )
    def k(ones_hbm, edges_hbm, zero_hbm, out_hbm, didx, ones_v, ssem, acc_sh):
        c = lax.axis_index("c")
        s = lax.axis_index("s")
        lo, m = _worker_span(c * NS + s)
        sl = pl.ds(s * RPT, RPT)
        pltpu.sync_copy(edges_hbm.at[1, pl.ds(lo, MAXC)], didx)
        pltpu.sync_copy(ones_hbm, ones_v)
        pltpu.sync_copy(zero_hbm.at[sl], acc_sh.at[sl])
        plsc.subcore_barrier()

        def group(jj, carry):
            for b in range(NB):
                j = jj * NB + b

                @pl.when(jj > 0)
                def _():
                    pltpu.make_async_copy(ones_v, acc_sh.at[didx.at[j]],
                                          ssem[b]).wait()

                @pl.when(j < m)
                def _():
                    pltpu.async_copy(ones_v, acc_sh.at[didx.at[j]], ssem[b],
                                     add=True)
            return carry

        lax.fori_loop(0, NG, group, 0)
        for b in range(NB):

            @pl.when((NG - 1) * NB + b < m)
            def _():
                pltpu.make_async_copy(ones_v, acc_sh.at[didx.at[b]],
                                      ssem[b]).wait()

        plsc.subcore_barrier()
        pltpu.sync_copy(acc_sh.at[sl], out_hbm.at[c, sl])

    return k


def _sc_layer1():
    F = 16
    mesh = plsc.VectorSubcoreMesh(core_axis_name="c", subcore_axis_name="s")

    @functools.partial(
        pl.kernel,
        mesh=mesh,
        out_type=(jax.ShapeDtypeStruct((NC, NP, F), jnp.float32),
                  jax.ShapeDtypeStruct((NP, F), jnp.float32),
                  jax.ShapeDtypeStruct((NP, F), jnp.float32)),
        compiler_params=_SC_PARAMS,
        scratch_types=[
            pltpu.VMEM((MAXC, CH), jnp.int32),
            pltpu.VMEM((MAXC, CH), jnp.int32),
            [pltpu.VMEM((CH, F), jnp.float32) for _ in range(NB)],
            [pltpu.SemaphoreType.DMA for _ in range(NB)],
            [pltpu.SemaphoreType.DMA for _ in range(NB)],
            pltpu.VMEM((RPT, F), jnp.float32),
            pltpu.VMEM((RPT, F), jnp.float32),
            pltpu.VMEM((RPT, F), jnp.float32),
            pltpu.VMEM((RPT, F), jnp.float32),
            pltpu.VMEM_SHARED((NP, F), jnp.float32),
            pltpu.VMEM_SHARED((NP, F), jnp.float32),
        ],
    )
    def k(degp_hbm, h1_hbm, edges_hbm, zero_hbm,
          out_hbm, g1_hbm, dinv_hbm,
          sidx, didx, rows, gsem, ssem, da_v, db_v, gg_v, dv_v, acc_sh, g_sh):
        c = lax.axis_index("c")
        s = lax.axis_index("s")
        lo, m = _worker_span(c * NS + s)
        sl = pl.ds(s * RPT, RPT)
        pltpu.sync_copy(edges_hbm.at[0, pl.ds(lo, MAXC)], sidx)
        pltpu.sync_copy(edges_hbm.at[1, pl.ds(lo, MAXC)], didx)
        pltpu.sync_copy(zero_hbm.at[sl], acc_sh.at[sl])
        pltpu.sync_copy(degp_hbm.at[0, sl], da_v)
        pltpu.sync_copy(degp_hbm.at[1, sl], db_v)
        pltpu.sync_copy(h1_hbm.at[sl], gg_v)

        def prep(r, carry):
            d = da_v[r] + db_v[r] + 1.0
            y = _rsqrt16(d)
            dv_v[r] = y
            gg_v[r] = y * gg_v[r]
            return carry

        lax.fori_loop(0, RPT, prep, 0)
        pltpu.sync_copy(gg_v, g_sh.at[sl])

        @pl.when(c == 0)
        def _():
            pltpu.sync_copy(gg_v, g1_hbm.at[sl])
            pltpu.sync_copy(dv_v, dinv_hbm.at[sl])

        plsc.subcore_barrier()
        _edge_pipeline(sidx, didx, rows, gsem, ssem, g_sh, acc_sh, m)
        plsc.subcore_barrier()
        pltpu.sync_copy(acc_sh.at[sl], out_hbm.at[c, sl])

    return k


def _sc_layer2():
    F = 16
    mesh = plsc.VectorSubcoreMesh(core_axis_name="c", subcore_axis_name="s")

    @functools.partial(
        pl.kernel,
        mesh=mesh,
        out_type=(jax.ShapeDtypeStruct((NC, NP, F), jnp.float32),
                  jax.ShapeDtypeStruct((NP, F), jnp.float32)),
        compiler_params=_SC_PARAMS,
        scratch_types=[
            pltpu.VMEM((MAXC, CH), jnp.int32),
            pltpu.VMEM((MAXC, CH), jnp.int32),
            [pltpu.VMEM((CH, F), jnp.float32) for _ in range(NB)],
            [pltpu.SemaphoreType.DMA for _ in range(NB)],
            [pltpu.SemaphoreType.DMA for _ in range(NB)],
            pltpu.VMEM((RPT, F), jnp.float32),
            pltpu.VMEM((RPT, F), jnp.float32),
            pltpu.VMEM((RPT, F), jnp.float32),
            pltpu.VMEM((RPT, F), jnp.float32),
            pltpu.VMEM((F,), jnp.float32),
            pltpu.VMEM_SHARED((NP, F), jnp.float32),
            pltpu.VMEM_SHARED((NP, F), jnp.float32),
        ],
    )
    def k(p1p_hbm, g1_hbm, dinv_hbm, b1_hbm, edges_hbm, zero_hbm,
          out_hbm, g2_hbm, sidx, didx, rows, gsem, ssem,
          pa_v, pb_v, gg_v, dv_v, b1_v, acc_sh, g_sh):
        c = lax.axis_index("c")
        s = lax.axis_index("s")
        lo, m = _worker_span(c * NS + s)
        sl = pl.ds(s * RPT, RPT)
        pltpu.sync_copy(edges_hbm.at[0, pl.ds(lo, MAXC)], sidx)
        pltpu.sync_copy(edges_hbm.at[1, pl.ds(lo, MAXC)], didx)
        pltpu.sync_copy(zero_hbm.at[sl], acc_sh.at[sl])
        pltpu.sync_copy(p1p_hbm.at[0, sl], pa_v)
        pltpu.sync_copy(p1p_hbm.at[1, sl], pb_v)
        pltpu.sync_copy(g1_hbm.at[sl], gg_v)
        pltpu.sync_copy(dinv_hbm.at[sl], dv_v)
        pltpu.sync_copy(b1_hbm, b1_v)
        b1r = b1_v[...]

        def combine(r, carry):
            dv = dv_v[r]
            z = jnp.maximum(dv * (pa_v[r] + pb_v[r] + gg_v[r]) + b1r, 0.0)
            gg_v[r] = dv * z
            return carry

        lax.fori_loop(0, RPT, combine, 0)
        pltpu.sync_copy(gg_v, g_sh.at[sl])

        @pl.when(c == 0)
        def _():
            pltpu.sync_copy(gg_v, g2_hbm.at[sl])

        plsc.subcore_barrier()
        _edge_pipeline(sidx, didx, rows, gsem, ssem, g_sh, acc_sh, m)
        plsc.subcore_barrier()
        pltpu.sync_copy(acc_sh.at[sl], out_hbm.at[c, sl])

    return k


def _sc_scatter(F):
    mesh = plsc.VectorSubcoreMesh(core_axis_name="c", subcore_axis_name="s")

    @functools.partial(
        pl.kernel,
        mesh=mesh,
        out_type=jax.ShapeDtypeStruct((NC, NP, F), jnp.float32),
        compiler_params=_SC_PARAMS,
        scratch_types=[
            pltpu.VMEM((MAXC, CH), jnp.int32),
            pltpu.VMEM((MAXC, CH), jnp.int32),
            [pltpu.VMEM((CH, F), jnp.float32) for _ in range(NB)],
            [pltpu.SemaphoreType.DMA for _ in range(NB)],
            [pltpu.SemaphoreType.DMA for _ in range(NB)],
            pltpu.VMEM_SHARED((NP, F), jnp.float32),
            pltpu.VMEM_SHARED((NP, F), jnp.float32),
        ],
    )
    def k(g_hbm, edges_hbm, zero_hbm, out_hbm,
          sidx, didx, rows, gsem, ssem, acc_sh, g_sh):
        c = lax.axis_index("c")
        s = lax.axis_index("s")
        lo, m = _worker_span(c * NS + s)
        sl = pl.ds(s * RPT, RPT)
        pltpu.sync_copy(edges_hbm.at[0, pl.ds(lo, MAXC)], sidx)
        pltpu.sync_copy(edges_hbm.at[1, pl.ds(lo, MAXC)], didx)
        pltpu.sync_copy(zero_hbm.at[sl], acc_sh.at[sl])
        pltpu.sync_copy(g_hbm.at[sl], g_sh.at[sl])
        plsc.subcore_barrier()
        _edge_pipeline(sidx, didx, rows, gsem, ssem, g_sh, acc_sh, m)
        plsc.subcore_barrier()
        pltpu.sync_copy(acc_sh.at[sl], out_hbm.at[c, sl])

    return k


def _tc_h1(x, w1):
    def body(x_ref, w_ref, h_ref):
        h = jnp.dot(x_ref[...], w_ref[...], preferred_element_type=jnp.float32)
        h_ref[...] = jnp.concatenate(
            [h, jnp.zeros((NP - N, 16), jnp.float32)], axis=0)

    return pl.pallas_call(
        body,
        out_shape=jax.ShapeDtypeStruct((NP, 16), jnp.float32),
    )(x, w1)


def _tc_mid3(pp, g, dinv, b, w2, w3):
    def body(pp_ref, g_ref, dinv_ref, b_ref, w2_ref, w3_ref, out_ref):
        dinv1 = dinv_ref[:, :1]
        h2 = jnp.dot(pp_ref[0] + pp_ref[1] + g_ref[...], w2_ref[...],
                     preferred_element_type=jnp.float32)
        z = jnp.maximum(dinv1 * h2 + b_ref[...], 0.0)
        out_ref[...] = dinv1 * jnp.dot(z, w3_ref[...],
                                       preferred_element_type=jnp.float32)

    return pl.pallas_call(
        body,
        out_shape=jax.ShapeDtypeStruct((NP, 16), jnp.float32),
    )(pp, g, dinv, b, w2, w3)


def _tc_final(pp, g, dinv, b, wc, bc):
    def body(pp_ref, g_ref, dinv_ref, b_ref, wc_ref, bc_ref, out_ref):
        dinv1 = dinv_ref[:, :1]
        z = jnp.maximum(dinv1 * (pp_ref[0] + pp_ref[1] + g_ref[...]) + b_ref[...], 0.0)
        res = jnp.dot(z, wc_ref[...],
                      preferred_element_type=jnp.float32) + bc_ref[...]
        out_ref[...] = res[:N]

    return pl.pallas_call(
        body,
        out_shape=jax.ShapeDtypeStruct((N, 3), jnp.float32),
    )(pp, g, dinv, b, wc, bc)


def kernel(x, edge_index, W1, b1, W2, b2, W3, b3, Wc, bc):
    edges = edge_index.astype(jnp.int32).reshape(2, RW, CH)

    zeros16 = jnp.zeros((NP, 16), jnp.float32)
    ones16 = jnp.ones((CH, 16), jnp.float32)

    h1 = _tc_h1(x, W1)
    degp = _sc_degree()(ones16, edges, zeros16)
    p1, g1, dinv = _sc_layer1()(degp, h1, edges, zeros16)
    p2, g2 = _sc_layer2()(p1, g1, dinv, b1, edges, zeros16)
    g3 = _tc_mid3(p2, g2, dinv, b2.reshape(1, 32), W2, W3)
    p3 = _sc_scatter(16)(g3, edges, zeros16)
    out = _tc_final(p3, g3, dinv, b3.reshape(1, 16), Wc, bc.reshape(1, 3))
    return out


# speedup vs baseline: 1.0337x; 1.0337x over previous
import functools

import jax
import jax.numpy as jnp
from jax import lax
from jax.experimental import pallas as pl
from jax.experimental.pallas import tpu as pltpu
from jax.experimental.pallas import tpu_sc as plsc

N = 10000
NP = 10240
E = 320000
NC = 2
NS = 16
NW = NC * NS
CH = 128
RW = E // CH
NB = 8
MAXC = RW // NW + 1
NG = 10
RPT = NP // NS

_SC_PARAMS = pltpu.CompilerParams(use_tc_tiling_on_sc=False,
                                  needs_layout_passes=False)


def _worker_span(wid):
    lo = wid * RW // NW
    hi = (wid + 1) * RW // NW
    return lo, hi - lo


def _rsqrt16(d):
    i = plsc.bitcast(d, jnp.int32)
    y = plsc.bitcast(0x5F3759DF - lax.shift_right_logical(i, 1), jnp.float32)
    for _ in range(3):
        y = y * (1.5 - 0.5 * d * y * y)
    return y


def _edge_pipeline(sidx, didx, rows, gsem, ssem, g_sh, acc_sh, m):
    for b in range(NB):
        pltpu.async_copy(g_sh.at[sidx.at[b]], rows[b], gsem[b])

    def group(jj, carry):
        for b in range(NB):
            j = jj * NB + b

            @pl.when(j < m)
            def _():
                pltpu.make_async_copy(g_sh.at[sidx.at[j]], rows[b],
                                      gsem[b]).wait()
                pltpu.sync_copy(rows[b], acc_sh.at[didx.at[j]], add=True)

            @pl.when(j + NB < m)
            def _():
                pltpu.async_copy(g_sh.at[sidx.at[j + NB]], rows[b],
                                 gsem[b])
        return carry

    lax.fori_loop(0, NG, group, 0)


def _sc_degree():
    mesh = plsc.VectorSubcoreMesh(core_axis_name="c", subcore_axis_name="s")

    @functools.partial(
        pl.kernel,
        mesh=mesh,
        out_type=jax.ShapeDtypeStruct((NC, NP, 16), jnp.float32),
        compiler_params=_SC_PARAMS,
        scratch_types=[
            pltpu.VMEM((MAXC, CH), jnp.int32),
            pltpu.VMEM((CH, 16), jnp.float32),
            [pltpu.SemaphoreType.DMA for _ in range(NB)],
            pltpu.VMEM_SHARED((NP, 16), jnp.float32),
        ],
    )
    def k(ones_hbm, edges_hbm, zero_hbm, out_hbm, didx, ones_v, ssem, acc_sh):
        c = lax.axis_index("c")
        s = lax.axis_index("s")
        lo, m = _worker_span(c * NS + s)
        sl = pl.ds(s * RPT, RPT)
        pltpu.sync_copy(edges_hbm.at[1, pl.ds(lo, MAXC)], didx)
        pltpu.sync_copy(ones_hbm, ones_v)
        pltpu.sync_copy(zero_hbm.at[sl], acc_sh.at[sl])
        plsc.subcore_barrier()

        def group(jj, carry):
            for b in range(NB):
                j = jj * NB + b

                @pl.when(jj > 0)
                def _():
                    pltpu.make_async_copy(ones_v, acc_sh.at[didx.at[j]],
                                          ssem[b]).wait()

                @pl.when(j < m)
                def _():
                    pltpu.async_copy(ones_v, acc_sh.at[didx.at[j]], ssem[b],
                                     add=True)
            return carry

        lax.fori_loop(0, NG, group, 0)
        for b in range(NB):

            @pl.when((NG - 1) * NB + b < m)
            def _():
                pltpu.make_async_copy(ones_v, acc_sh.at[didx.at[b]],
                                      ssem[b]).wait()

        plsc.subcore_barrier()
        pltpu.sync_copy(acc_sh.at[sl], out_hbm.at[c, sl])

    return k


def _sc_layer1():
    F = 16
    mesh = plsc.VectorSubcoreMesh(core_axis_name="c", subcore_axis_name="s")

    @functools.partial(
        pl.kernel,
        mesh=mesh,
        out_type=(jax.ShapeDtypeStruct((NC, NP, F), jnp.float32),
                  jax.ShapeDtypeStruct((NP, F), jnp.float32),
                  jax.ShapeDtypeStruct((NP, F), jnp.float32)),
        compiler_params=_SC_PARAMS,
        scratch_types=[
            pltpu.VMEM((MAXC, CH), jnp.int32),
            pltpu.VMEM((MAXC, CH), jnp.int32),
            [pltpu.VMEM((CH, F), jnp.float32) for _ in range(NB)],
            [pltpu.SemaphoreType.DMA for _ in range(NB)],
            [pltpu.SemaphoreType.DMA for _ in range(NB)],
            pltpu.VMEM((RPT, F), jnp.float32),
            pltpu.VMEM((RPT, F), jnp.float32),
            pltpu.VMEM((RPT, F), jnp.float32),
            pltpu.VMEM((RPT, F), jnp.float32),
            pltpu.VMEM_SHARED((NP, F), jnp.float32),
            pltpu.VMEM_SHARED((NP, F), jnp.float32),
        ],
    )
    def k(degp_hbm, h1_hbm, edges_hbm, zero_hbm,
          out_hbm, g1_hbm, dinv_hbm,
          sidx, didx, rows, gsem, ssem, da_v, db_v, gg_v, dv_v, acc_sh, g_sh):
        c = lax.axis_index("c")
        s = lax.axis_index("s")
        lo, m = _worker_span(c * NS + s)
        sl = pl.ds(s * RPT, RPT)
        pltpu.sync_copy(edges_hbm.at[0, pl.ds(lo, MAXC)], sidx)
        pltpu.sync_copy(edges_hbm.at[1, pl.ds(lo, MAXC)], didx)
        pltpu.sync_copy(zero_hbm.at[sl], acc_sh.at[sl])
        pltpu.sync_copy(degp_hbm.at[0, sl], da_v)
        pltpu.sync_copy(degp_hbm.at[1, sl], db_v)
        pltpu.sync_copy(h1_hbm.at[sl], gg_v)

        def prep(r, carry):
            d = da_v[r] + db_v[r] + 1.0
            y = _rsqrt16(d)
            dv_v[r] = y
            gg_v[r] = y * gg_v[r]
            return carry

        lax.fori_loop(0, RPT, prep, 0)
        pltpu.sync_copy(gg_v, g_sh.at[sl])

        @pl.when(c == 0)
        def _():
            pltpu.sync_copy(gg_v, g1_hbm.at[sl])
            pltpu.sync_copy(dv_v, dinv_hbm.at[sl])

        plsc.subcore_barrier()
        _edge_pipeline(sidx, didx, rows, gsem, ssem, g_sh, acc_sh, m)
        plsc.subcore_barrier()
        pltpu.sync_copy(acc_sh.at[sl], out_hbm.at[c, sl])

    return k


def _sc_layer2():
    F = 16
    mesh = plsc.VectorSubcoreMesh(core_axis_name="c", subcore_axis_name="s")

    @functools.partial(
        pl.kernel,
        mesh=mesh,
        out_type=(jax.ShapeDtypeStruct((NC, NP, F), jnp.float32),
                  jax.ShapeDtypeStruct((NP, F), jnp.float32)),
        compiler_params=_SC_PARAMS,
        scratch_types=[
            pltpu.VMEM((MAXC, CH), jnp.int32),
            pltpu.VMEM((MAXC, CH), jnp.int32),
            [pltpu.VMEM((CH, F), jnp.float32) for _ in range(NB)],
            [pltpu.SemaphoreType.DMA for _ in range(NB)],
            [pltpu.SemaphoreType.DMA for _ in range(NB)],
            pltpu.VMEM((RPT, F), jnp.float32),
            pltpu.VMEM((RPT, F), jnp.float32),
            pltpu.VMEM((RPT, F), jnp.float32),
            pltpu.VMEM((RPT, F), jnp.float32),
            pltpu.VMEM((F,), jnp.float32),
            pltpu.VMEM_SHARED((NP, F), jnp.float32),
            pltpu.VMEM_SHARED((NP, F), jnp.float32),
        ],
    )
    def k(p1p_hbm, g1_hbm, dinv_hbm, b1_hbm, edges_hbm, zero_hbm,
          out_hbm, g2_hbm, sidx, didx, rows, gsem, ssem,
          pa_v, pb_v, gg_v, dv_v, b1_v, acc_sh, g_sh):
        c = lax.axis_index("c")
        s = lax.axis_index("s")
        lo, m = _worker_span(c * NS + s)
        sl = pl.ds(s * RPT, RPT)
        pltpu.sync_copy(edges_hbm.at[0, pl.ds(lo, MAXC)], sidx)
        pltpu.sync_copy(edges_hbm.at[1, pl.ds(lo, MAXC)], didx)
        pltpu.sync_copy(zero_hbm.at[sl], acc_sh.at[sl])
        pltpu.sync_copy(p1p_hbm.at[0, sl], pa_v)
        pltpu.sync_copy(p1p_hbm.at[1, sl], pb_v)
        pltpu.sync_copy(g1_hbm.at[sl], gg_v)
        pltpu.sync_copy(dinv_hbm.at[sl], dv_v)
        pltpu.sync_copy(b1_hbm, b1_v)
        b1r = b1_v[...]

        def combine(r, carry):
            dv = dv_v[r]
            z = jnp.maximum(dv * (pa_v[r] + pb_v[r] + gg_v[r]) + b1r, 0.0)
            gg_v[r] = dv * z
            return carry

        lax.fori_loop(0, RPT, combine, 0)
        pltpu.sync_copy(gg_v, g_sh.at[sl])

        @pl.when(c == 0)
        def _():
            pltpu.sync_copy(gg_v, g2_hbm.at[sl])

        plsc.subcore_barrier()
        _edge_pipeline(sidx, didx, rows, gsem, ssem, g_sh, acc_sh, m)
        plsc.subcore_barrier()
        pltpu.sync_copy(acc_sh.at[sl], out_hbm.at[c, sl])

    return k


def _sc_scatter(F):
    mesh = plsc.VectorSubcoreMesh(core_axis_name="c", subcore_axis_name="s")

    @functools.partial(
        pl.kernel,
        mesh=mesh,
        out_type=jax.ShapeDtypeStruct((NC, NP, F), jnp.float32),
        compiler_params=_SC_PARAMS,
        scratch_types=[
            pltpu.VMEM((MAXC, CH), jnp.int32),
            pltpu.VMEM((MAXC, CH), jnp.int32),
            [pltpu.VMEM((CH, F), jnp.float32) for _ in range(NB)],
            [pltpu.SemaphoreType.DMA for _ in range(NB)],
            [pltpu.SemaphoreType.DMA for _ in range(NB)],
            pltpu.VMEM_SHARED((NP, F), jnp.float32),
            pltpu.VMEM_SHARED((NP, F), jnp.float32),
        ],
    )
    def k(g_hbm, edges_hbm, zero_hbm, out_hbm,
          sidx, didx, rows, gsem, ssem, acc_sh, g_sh):
        c = lax.axis_index("c")
        s = lax.axis_index("s")
        lo, m = _worker_span(c * NS + s)
        sl = pl.ds(s * RPT, RPT)
        pltpu.sync_copy(edges_hbm.at[0, pl.ds(lo, MAXC)], sidx)
        pltpu.sync_copy(edges_hbm.at[1, pl.ds(lo, MAXC)], didx)
        pltpu.sync_copy(zero_hbm.at[sl], acc_sh.at[sl])
        pltpu.sync_copy(g_hbm.at[sl], g_sh.at[sl])
        plsc.subcore_barrier()
        _edge_pipeline(sidx, didx, rows, gsem, ssem, g_sh, acc_sh, m)
        plsc.subcore_barrier()
        pltpu.sync_copy(acc_sh.at[sl], out_hbm.at[c, sl])

    return k


def _tc_h1(x, w1):
    def body(x_ref, w_ref, h_ref):
        h = jnp.dot(x_ref[...], w_ref[...], preferred_element_type=jnp.float32)
        h_ref[...] = jnp.concatenate(
            [h, jnp.zeros((NP - N, 16), jnp.float32)], axis=0)

    return pl.pallas_call(
        body,
        out_shape=jax.ShapeDtypeStruct((NP, 16), jnp.float32),
    )(x, w1)


def _tc_mid3(pp, g, dinv, b, w2, w3):
    def body(pp_ref, g_ref, dinv_ref, b_ref, w2_ref, w3_ref, out_ref):
        dinv1 = dinv_ref[:, :1]
        h2 = jnp.dot(pp_ref[0] + pp_ref[1] + g_ref[...], w2_ref[...],
                     preferred_element_type=jnp.float32)
        z = jnp.maximum(dinv1 * h2 + b_ref[...], 0.0)
        out_ref[...] = dinv1 * jnp.dot(z, w3_ref[...],
                                       preferred_element_type=jnp.float32)

    return pl.pallas_call(
        body,
        out_shape=jax.ShapeDtypeStruct((NP, 16), jnp.float32),
    )(pp, g, dinv, b, w2, w3)


def _tc_final(pp, g, dinv, b, wc, bc):
    def body(pp_ref, g_ref, dinv_ref, b_ref, wc_ref, bc_ref, out_ref):
        dinv1 = dinv_ref[:, :1]
        z = jnp.maximum(dinv1 * (pp_ref[0] + pp_ref[1] + g_ref[...]) + b_ref[...], 0.0)
        res = jnp.dot(z, wc_ref[...],
                      preferred_element_type=jnp.float32) + bc_ref[...]
        out_ref[...] = res[:N]

    return pl.pallas_call(
        body,
        out_shape=jax.ShapeDtypeStruct((N, 3), jnp.float32),
    )(pp, g, dinv, b, wc, bc)


def kernel(x, edge_index, W1, b1, W2, b2, W3, b3, Wc, bc):
    edges = edge_index.astype(jnp.int32).reshape(2, RW, CH)

    zeros16 = jnp.zeros((NP, 16), jnp.float32)
    ones16 = jnp.ones((CH, 16), jnp.float32)

    h1 = _tc_h1(x, W1)
    degp = _sc_degree()(ones16, edges, zeros16)
    p1, g1, dinv = _sc_layer1()(degp, h1, edges, zeros16)
    p2, g2 = _sc_layer2()(p1, g1, dinv, b1, edges, zeros16)
    g3 = _tc_mid3(p2, g2, dinv, b2.reshape(1, 32), W2, W3)
    p3 = _sc_scatter(16)(g3, edges, zeros16)
    out = _tc_final(p3, g3, dinv, b3.reshape(1, 16), Wc, bc.reshape(1, 3))
    return out


# speedup vs baseline: 1.1268x; 1.0900x over previous
import functools

import jax
import jax.numpy as jnp
from jax import lax
from jax.experimental import pallas as pl
from jax.experimental.pallas import tpu as pltpu
from jax.experimental.pallas import tpu_sc as plsc

N = 10000
NP = 10240
E = 320000
NC = 2
NS = 16
NW = NC * NS
CH = 128
RW = E // CH
NB = 4
MAXC = RW // NW + 1
NG = 20
RPT = NP // NS

_SC_PARAMS = pltpu.CompilerParams(use_tc_tiling_on_sc=False,
                                  needs_layout_passes=False)


def _worker_span(wid):
    lo = wid * RW // NW
    hi = (wid + 1) * RW // NW
    return lo, hi - lo


def _rsqrt16(d):
    i = plsc.bitcast(d, jnp.int32)
    y = plsc.bitcast(0x5F3759DF - lax.shift_right_logical(i, 1), jnp.float32)
    for _ in range(3):
        y = y * (1.5 - 0.5 * d * y * y)
    return y


def _edge_pipeline(sidx, didx, rows, gsem, ssem, g_sh, acc_sh, m):
    for b in range(NB):
        pltpu.async_copy(g_sh.at[sidx.at[b]], rows[b], gsem[b])

    def group(jj, carry):
        for b in range(NB):
            j = jj * NB + b

            @pl.when(j < m)
            def _():
                pltpu.make_async_copy(g_sh.at[sidx.at[j]], rows[b],
                                      gsem[b]).wait()
                pltpu.sync_copy(rows[b], acc_sh.at[didx.at[j]], add=True)

            @pl.when(j + NB < m)
            def _():
                pltpu.async_copy(g_sh.at[sidx.at[j + NB]], rows[b],
                                 gsem[b])
        return carry

    lax.fori_loop(0, NG, group, 0)


def _sc_degree():
    mesh = plsc.VectorSubcoreMesh(core_axis_name="c", subcore_axis_name="s")

    @functools.partial(
        pl.kernel,
        mesh=mesh,
        out_type=jax.ShapeDtypeStruct((NC, NP, 16), jnp.float32),
        compiler_params=_SC_PARAMS,
        scratch_types=[
            pltpu.VMEM((MAXC, CH), jnp.int32),
            pltpu.VMEM((CH, 16), jnp.float32),
            [pltpu.SemaphoreType.DMA for _ in range(NB)],
            pltpu.VMEM_SHARED((NP, 16), jnp.float32),
        ],
    )
    def k(ones_hbm, edges_hbm, zero_hbm, out_hbm, didx, ones_v, ssem, acc_sh):
        c = lax.axis_index("c")
        s = lax.axis_index("s")
        lo, m = _worker_span(c * NS + s)
        sl = pl.ds(s * RPT, RPT)
        stage = [
            pltpu.async_copy(edges_hbm.at[1, pl.ds(lo, MAXC)], didx, ssem[0]),
            pltpu.async_copy(ones_hbm, ones_v, ssem[1]),
            pltpu.async_copy(zero_hbm.at[sl], acc_sh.at[sl], ssem[2]),
        ]
        for d_ in stage:
            d_.wait()
        plsc.subcore_barrier()

        def group(jj, carry):
            for b in range(NB):
                j = jj * NB + b

                @pl.when(jj > 0)
                def _():
                    pltpu.make_async_copy(ones_v, acc_sh.at[didx.at[j]],
                                          ssem[b]).wait()

                @pl.when(j < m)
                def _():
                    pltpu.async_copy(ones_v, acc_sh.at[didx.at[j]], ssem[b],
                                     add=True)
            return carry

        lax.fori_loop(0, NG, group, 0)
        for b in range(NB):

            @pl.when((NG - 1) * NB + b < m)
            def _():
                pltpu.make_async_copy(ones_v, acc_sh.at[didx.at[b]],
                                      ssem[b]).wait()

        plsc.subcore_barrier()
        pltpu.sync_copy(acc_sh.at[sl], out_hbm.at[c, sl])

    return k


def _sc_layer1():
    F = 16
    mesh = plsc.VectorSubcoreMesh(core_axis_name="c", subcore_axis_name="s")

    @functools.partial(
        pl.kernel,
        mesh=mesh,
        out_type=(jax.ShapeDtypeStruct((NC, NP, F), jnp.float32),
                  jax.ShapeDtypeStruct((NP, F), jnp.float32),
                  jax.ShapeDtypeStruct((NP, F), jnp.float32)),
        compiler_params=_SC_PARAMS,
        scratch_types=[
            pltpu.VMEM((MAXC, CH), jnp.int32),
            pltpu.VMEM((MAXC, CH), jnp.int32),
            [pltpu.VMEM((CH, F), jnp.float32) for _ in range(NB)],
            [pltpu.SemaphoreType.DMA for _ in range(NB)],
            [pltpu.SemaphoreType.DMA for _ in range(NB)],
            pltpu.VMEM((RPT, F), jnp.float32),
            pltpu.VMEM((RPT, F), jnp.float32),
            pltpu.VMEM((RPT, F), jnp.float32),
            pltpu.VMEM((RPT, F), jnp.float32),
            pltpu.VMEM_SHARED((NP, F), jnp.float32),
            pltpu.VMEM_SHARED((NP, F), jnp.float32),
        ],
    )
    def k(degp_hbm, h1_hbm, edges_hbm, zero_hbm,
          out_hbm, g1_hbm, dinv_hbm,
          sidx, didx, rows, gsem, ssem, da_v, db_v, gg_v, dv_v, acc_sh, g_sh):
        c = lax.axis_index("c")
        s = lax.axis_index("s")
        lo, m = _worker_span(c * NS + s)
        sl = pl.ds(s * RPT, RPT)
        stage = [
            pltpu.async_copy(edges_hbm.at[0, pl.ds(lo, MAXC)], sidx, gsem[0]),
            pltpu.async_copy(edges_hbm.at[1, pl.ds(lo, MAXC)], didx, gsem[1]),
            pltpu.async_copy(zero_hbm.at[sl], acc_sh.at[sl], gsem[2]),
            pltpu.async_copy(degp_hbm.at[0, sl], da_v, gsem[3]),
            pltpu.async_copy(degp_hbm.at[1, sl], db_v, ssem[0]),
            pltpu.async_copy(h1_hbm.at[sl], gg_v, ssem[1]),
        ]
        for d_ in stage:
            d_.wait()

        def prep(r2, carry):
            for u in range(2):
                r = r2 * 2 + u
                d = da_v[r] + db_v[r] + 1.0
                y = _rsqrt16(d)
                dv_v[r] = y
                gg_v[r] = y * gg_v[r]
            return carry

        lax.fori_loop(0, RPT // 2, prep, 0)
        pltpu.sync_copy(gg_v, g_sh.at[sl])

        @pl.when(c == 0)
        def _():
            pltpu.sync_copy(gg_v, g1_hbm.at[sl])
            pltpu.sync_copy(dv_v, dinv_hbm.at[sl])

        plsc.subcore_barrier()
        _edge_pipeline(sidx, didx, rows, gsem, ssem, g_sh, acc_sh, m)
        plsc.subcore_barrier()
        pltpu.sync_copy(acc_sh.at[sl], out_hbm.at[c, sl])

    return k


def _sc_layer2():
    F = 16
    mesh = plsc.VectorSubcoreMesh(core_axis_name="c", subcore_axis_name="s")

    @functools.partial(
        pl.kernel,
        mesh=mesh,
        out_type=(jax.ShapeDtypeStruct((NC, NP, F), jnp.float32),
                  jax.ShapeDtypeStruct((NP, F), jnp.float32)),
        compiler_params=_SC_PARAMS,
        scratch_types=[
            pltpu.VMEM((MAXC, CH), jnp.int32),
            pltpu.VMEM((MAXC, CH), jnp.int32),
            [pltpu.VMEM((CH, F), jnp.float32) for _ in range(NB)],
            [pltpu.SemaphoreType.DMA for _ in range(NB)],
            [pltpu.SemaphoreType.DMA for _ in range(NB)],
            pltpu.VMEM((RPT, F), jnp.float32),
            pltpu.VMEM((RPT, F), jnp.float32),
            pltpu.VMEM((RPT, F), jnp.float32),
            pltpu.VMEM((RPT, F), jnp.float32),
            pltpu.VMEM((F,), jnp.float32),
            pltpu.VMEM_SHARED((NP, F), jnp.float32),
            pltpu.VMEM_SHARED((NP, F), jnp.float32),
        ],
    )
    def k(p1p_hbm, g1_hbm, dinv_hbm, b1_hbm, edges_hbm, zero_hbm,
          out_hbm, g2_hbm, sidx, didx, rows, gsem, ssem,
          pa_v, pb_v, gg_v, dv_v, b1_v, acc_sh, g_sh):
        c = lax.axis_index("c")
        s = lax.axis_index("s")
        lo, m = _worker_span(c * NS + s)
        sl = pl.ds(s * RPT, RPT)
        stage = [
            pltpu.async_copy(edges_hbm.at[0, pl.ds(lo, MAXC)], sidx, gsem[0]),
            pltpu.async_copy(edges_hbm.at[1, pl.ds(lo, MAXC)], didx, gsem[1]),
            pltpu.async_copy(zero_hbm.at[sl], acc_sh.at[sl], gsem[2]),
            pltpu.async_copy(p1p_hbm.at[0, sl], pa_v, gsem[3]),
            pltpu.async_copy(p1p_hbm.at[1, sl], pb_v, ssem[0]),
            pltpu.async_copy(g1_hbm.at[sl], gg_v, ssem[1]),
            pltpu.async_copy(dinv_hbm.at[sl], dv_v, ssem[2]),
            pltpu.async_copy(b1_hbm, b1_v, ssem[3]),
        ]
        for d_ in stage:
            d_.wait()
        b1r = b1_v[...]

        def combine(r2, carry):
            for u in range(2):
                r = r2 * 2 + u
                dv = dv_v[r]
                z = jnp.maximum(dv * (pa_v[r] + pb_v[r] + gg_v[r]) + b1r, 0.0)
                gg_v[r] = dv * z
            return carry

        lax.fori_loop(0, RPT // 2, combine, 0)
        pltpu.sync_copy(gg_v, g_sh.at[sl])

        @pl.when(c == 0)
        def _():
            pltpu.sync_copy(gg_v, g2_hbm.at[sl])

        plsc.subcore_barrier()
        _edge_pipeline(sidx, didx, rows, gsem, ssem, g_sh, acc_sh, m)
        plsc.subcore_barrier()
        pltpu.sync_copy(acc_sh.at[sl], out_hbm.at[c, sl])

    return k


def _sc_scatter(F):
    mesh = plsc.VectorSubcoreMesh(core_axis_name="c", subcore_axis_name="s")

    @functools.partial(
        pl.kernel,
        mesh=mesh,
        out_type=jax.ShapeDtypeStruct((NC, NP, F), jnp.float32),
        compiler_params=_SC_PARAMS,
        scratch_types=[
            pltpu.VMEM((MAXC, CH), jnp.int32),
            pltpu.VMEM((MAXC, CH), jnp.int32),
            [pltpu.VMEM((CH, F), jnp.float32) for _ in range(NB)],
            [pltpu.SemaphoreType.DMA for _ in range(NB)],
            [pltpu.SemaphoreType.DMA for _ in range(NB)],
            pltpu.VMEM_SHARED((NP, F), jnp.float32),
            pltpu.VMEM_SHARED((NP, F), jnp.float32),
        ],
    )
    def k(g_hbm, edges_hbm, zero_hbm, out_hbm,
          sidx, didx, rows, gsem, ssem, acc_sh, g_sh):
        c = lax.axis_index("c")
        s = lax.axis_index("s")
        lo, m = _worker_span(c * NS + s)
        sl = pl.ds(s * RPT, RPT)
        stage = [
            pltpu.async_copy(edges_hbm.at[0, pl.ds(lo, MAXC)], sidx, gsem[0]),
            pltpu.async_copy(edges_hbm.at[1, pl.ds(lo, MAXC)], didx, gsem[1]),
            pltpu.async_copy(zero_hbm.at[sl], acc_sh.at[sl], gsem[2]),
            pltpu.async_copy(g_hbm.at[sl], g_sh.at[sl], gsem[3]),
        ]
        for d_ in stage:
            d_.wait()
        plsc.subcore_barrier()
        _edge_pipeline(sidx, didx, rows, gsem, ssem, g_sh, acc_sh, m)
        plsc.subcore_barrier()
        pltpu.sync_copy(acc_sh.at[sl], out_hbm.at[c, sl])

    return k


def _tc_h1(x, w1):
    def body(x_ref, w_ref, h_ref):
        h = jnp.dot(x_ref[...], w_ref[...], preferred_element_type=jnp.float32)
        h_ref[...] = jnp.concatenate(
            [h, jnp.zeros((NP - N, 16), jnp.float32)], axis=0)

    return pl.pallas_call(
        body,
        out_shape=jax.ShapeDtypeStruct((NP, 16), jnp.float32),
    )(x, w1)


def _tc_mid3(pp, g, dinv, b, w2, w3):
    def body(pp_ref, g_ref, dinv_ref, b_ref, w2_ref, w3_ref, out_ref):
        dinv1 = dinv_ref[:, :1]
        h2 = jnp.dot(pp_ref[0] + pp_ref[1] + g_ref[...], w2_ref[...],
                     preferred_element_type=jnp.float32)
        z = jnp.maximum(dinv1 * h2 + b_ref[...], 0.0)
        out_ref[...] = dinv1 * jnp.dot(z, w3_ref[...],
                                       preferred_element_type=jnp.float32)

    return pl.pallas_call(
        body,
        out_shape=jax.ShapeDtypeStruct((NP, 16), jnp.float32),
    )(pp, g, dinv, b, w2, w3)


def _tc_final(pp, g, dinv, b, wc, bc):
    def body(pp_ref, g_ref, dinv_ref, b_ref, wc_ref, bc_ref, out_ref):
        dinv1 = dinv_ref[:, :1]
        z = jnp.maximum(dinv1 * (pp_ref[0] + pp_ref[1] + g_ref[...]) + b_ref[...], 0.0)
        res = jnp.dot(z, wc_ref[...],
                      preferred_element_type=jnp.float32) + bc_ref[...]
        out_ref[...] = res[:N]

    return pl.pallas_call(
        body,
        out_shape=jax.ShapeDtypeStruct((N, 3), jnp.float32),
    )(pp, g, dinv, b, wc, bc)


def kernel(x, edge_index, W1, b1, W2, b2, W3, b3, Wc, bc):
    edges = edge_index.astype(jnp.int32).reshape(2, RW, CH)

    zeros16 = jnp.zeros((NP, 16), jnp.float32)
    ones16 = jnp.ones((CH, 16), jnp.float32)

    h1 = _tc_h1(x, W1)
    degp = _sc_degree()(ones16, edges, zeros16)
    p1, g1, dinv = _sc_layer1()(degp, h1, edges, zeros16)
    p2, g2 = _sc_layer2()(p1, g1, dinv, b1, edges, zeros16)
    g3 = _tc_mid3(p2, g2, dinv, b2.reshape(1, 32), W2, W3)
    p3 = _sc_scatter(16)(g3, edges, zeros16)
    out = _tc_final(p3, g3, dinv, b3.reshape(1, 16), Wc, bc.reshape(1, 3))
    return out
